# Initial kernel scaffold; baseline (speedup 1.0000x reference)
import jax
import jax.numpy as jnp
from jax.experimental import pallas as pl


def kernel(h, sigma, keys):
    raise NotImplementedError("write your pallas kernel here")


# speedup vs baseline: 1.6672x; 1.6672x over previous
import functools

import jax
import jax.numpy as jnp
from jax import lax
from jax.experimental import pallas as pl

B, K, D_H, NUM_AGENTS = 4096, 100, 128, 1000
A_PAD = 1024
ROWS = 800
SAMPLES = ROWS // K
GRID = (B * K) // ROWS


def _body(hf_ref, sig_ref, keys_ref, out_ref):
    h = hf_ref[...]
    sig = sig_ref[...]

    agent_iota = lax.broadcasted_iota(jnp.int32, (ROWS, A_PAD), 1)
    oh = (sig == agent_iota).astype(jnp.bfloat16)
    g = jnp.dot(oh, keys_ref[...], preferred_element_type=jnp.float32)

    logits = jnp.sum(g * h, axis=1, keepdims=True)
    e = jnp.exp(logits)

    bi = lax.broadcasted_iota(jnp.int32, (SAMPLES, ROWS), 0)
    mi = lax.broadcasted_iota(jnp.int32, (SAMPLES, ROWS), 1)
    pt = ((mi >= bi * K) & (mi < (bi + 1) * K)).astype(jnp.float32)
    bi2 = lax.broadcasted_iota(jnp.int32, (ROWS, SAMPLES), 1)
    mi2 = lax.broadcasted_iota(jnp.int32, (ROWS, SAMPLES), 0)
    p = ((mi2 >= bi2 * K) & (mi2 < (bi2 + 1) * K)).astype(jnp.float32)

    z = jnp.dot(pt, e, preferred_element_type=jnp.float32)
    row_z = jnp.dot(p, z, preferred_element_type=jnp.float32)
    alpha = e / row_z

    out_ref[...] = jnp.dot(pt, alpha * h, preferred_element_type=jnp.float32)


@jax.jit
def kernel(h, sigma, keys):
    hf = h.reshape(B * K, D_H)
    sigc = sigma.astype(jnp.int32).reshape(B * K, 1)
    keys_pad = jnp.zeros((A_PAD, D_H), jnp.bfloat16).at[:NUM_AGENTS].set(
        keys.astype(jnp.bfloat16))

    return pl.pallas_call(
        _body,
        grid=(GRID,),
        in_specs=[
            pl.BlockSpec((ROWS, D_H), lambda i: (i, 0)),
            pl.BlockSpec((ROWS, 1), lambda i: (i, 0)),
            pl.BlockSpec((A_PAD, D_H), lambda i: (0, 0)),
        ],
        out_specs=pl.BlockSpec((SAMPLES, D_H), lambda i: (i, 0)),
        out_shape=jax.ShapeDtypeStruct((B, D_H), jnp.float32),
    )(hf, sigc, keys_pad)


# speedup vs baseline: 1.8997x; 1.1395x over previous
import functools

import jax
import jax.numpy as jnp
from jax import lax
from jax.experimental import pallas as pl

B, K, D_H, NUM_AGENTS = 4096, 100, 128, 1000
A_PAD = 1024
ROWS = 800
SAMPLES = ROWS // K
GRID = (B * K) // ROWS


def _body(hf_ref, sig_ref, keys_ref, out_ref):
    h = hf_ref[...]
    sig = sig_ref[...]

    agent_iota = lax.broadcasted_iota(jnp.int32, (ROWS, A_PAD), 1)
    oh = (sig == agent_iota).astype(jnp.bfloat16)
    g = jnp.dot(oh, keys_ref[...], preferred_element_type=jnp.float32)

    logits = jnp.sum(g * h, axis=1, keepdims=True)
    e = jnp.exp(logits)

    bi = lax.broadcasted_iota(jnp.int32, (SAMPLES, ROWS), 0)
    mi = lax.broadcasted_iota(jnp.int32, (SAMPLES, ROWS), 1)
    pt = ((mi >= bi * K) & (mi < (bi + 1) * K)).astype(jnp.float32)

    z = jnp.dot(pt, e, preferred_element_type=jnp.float32)
    num = jnp.dot(pt, e * h, preferred_element_type=jnp.float32)
    out_ref[...] = num / z


@jax.jit
def kernel(h, sigma, keys):
    hf = h.reshape(B * K, D_H)
    sigc = sigma.astype(jnp.int32).reshape(B * K, 1)
    keys_pad = jnp.zeros((A_PAD, D_H), jnp.bfloat16).at[:NUM_AGENTS].set(
        keys.astype(jnp.bfloat16))

    return pl.pallas_call(
        _body,
        grid=(GRID,),
        in_specs=[
            pl.BlockSpec((ROWS, D_H), lambda i: (i, 0)),
            pl.BlockSpec((ROWS, 1), lambda i: (i, 0)),
            pl.BlockSpec((A_PAD, D_H), lambda i: (0, 0)),
        ],
        out_specs=pl.BlockSpec((SAMPLES, D_H), lambda i: (i, 0)),
        out_shape=jax.ShapeDtypeStruct((B, D_H), jnp.float32),
    )(hf, sigc, keys_pad)


# speedup vs baseline: 2.2878x; 1.2043x over previous
import functools

import jax
import jax.numpy as jnp
from jax import lax
from jax.experimental import pallas as pl

B, K, D_H, NUM_AGENTS = 4096, 100, 128, 1000
A_PAD = 1024
ROWS = 1600
SAMPLES = ROWS // K
GRID = (B * K) // ROWS


def _body(hf_ref, sig_ref, keys_ref, out_ref):
    h = hf_ref[...]
    sig = sig_ref[...]

    agent_iota = lax.broadcasted_iota(jnp.int32, (ROWS, A_PAD), 1)
    oh = (sig == agent_iota).astype(jnp.bfloat16)
    g = jnp.dot(oh, keys_ref[...], preferred_element_type=jnp.float32)

    logits = jnp.sum(g * h, axis=1, keepdims=True)
    e = jnp.exp(logits)

    bi = lax.broadcasted_iota(jnp.int32, (SAMPLES, ROWS), 0)
    mi = lax.broadcasted_iota(jnp.int32, (SAMPLES, ROWS), 1)
    pt = ((mi >= bi * K) & (mi < (bi + 1) * K)).astype(jnp.float32)

    z = jnp.dot(pt, e, preferred_element_type=jnp.float32)
    num = jnp.dot(pt, e * h, preferred_element_type=jnp.float32)
    out_ref[...] = num / z


@jax.jit
def kernel(h, sigma, keys):
    hf = h.reshape(B * K, D_H)
    sigc = sigma.astype(jnp.int32).reshape(B * K, 1)
    keys_pad = jnp.zeros((A_PAD, D_H), jnp.bfloat16).at[:NUM_AGENTS].set(
        keys.astype(jnp.bfloat16))

    return pl.pallas_call(
        _body,
        grid=(GRID,),
        in_specs=[
            pl.BlockSpec((ROWS, D_H), lambda i: (i, 0)),
            pl.BlockSpec((ROWS, 1), lambda i: (i, 0)),
            pl.BlockSpec((A_PAD, D_H), lambda i: (0, 0)),
        ],
        out_specs=pl.BlockSpec((SAMPLES, D_H), lambda i: (i, 0)),
        out_shape=jax.ShapeDtypeStruct((B, D_H), jnp.float32),
    )(hf, sigc, keys_pad)


# speedup vs baseline: 2.5116x; 1.0978x over previous
import functools

import jax
import jax.numpy as jnp
from jax import lax
from jax.experimental import pallas as pl

B, K, D_H, NUM_AGENTS = 4096, 100, 128, 1000
A_PAD = 1024
ROWS = 6400
SAMPLES = ROWS // K
GRID = (B * K) // ROWS


def _body(hf_ref, sig_ref, keys_ref, out_ref):
    h = hf_ref[...]
    sig = sig_ref[...]

    agent_iota = lax.broadcasted_iota(jnp.int32, (ROWS, A_PAD), 1)
    oh = (sig == agent_iota).astype(jnp.bfloat16)
    g = jnp.dot(oh, keys_ref[...], preferred_element_type=jnp.float32)

    logits = jnp.sum(g * h, axis=1, keepdims=True)
    e = jnp.exp(logits)

    bi = lax.broadcasted_iota(jnp.int32, (SAMPLES, ROWS), 0)
    mi = lax.broadcasted_iota(jnp.int32, (SAMPLES, ROWS), 1)
    seg = (mi >= bi * K) & (mi < (bi + 1) * K)
    pt = seg.astype(jnp.float32)
    ptb = seg.astype(jnp.bfloat16)

    z = jnp.dot(pt, e, preferred_element_type=jnp.float32)
    eh = (e * h).astype(jnp.bfloat16)
    num = jnp.dot(ptb, eh, preferred_element_type=jnp.float32)
    out_ref[...] = num / z


@jax.jit
def kernel(h, sigma, keys):
    hf = h.reshape(B * K, D_H)
    sigc = sigma.astype(jnp.int32).reshape(B * K, 1)
    keys_pad = jnp.zeros((A_PAD, D_H), jnp.bfloat16).at[:NUM_AGENTS].set(
        keys.astype(jnp.bfloat16))

    return pl.pallas_call(
        _body,
        grid=(GRID,),
        in_specs=[
            pl.BlockSpec((ROWS, D_H), lambda i: (i, 0)),
            pl.BlockSpec((ROWS, 1), lambda i: (i, 0)),
            pl.BlockSpec((A_PAD, D_H), lambda i: (0, 0)),
        ],
        out_specs=pl.BlockSpec((SAMPLES, D_H), lambda i: (i, 0)),
        out_shape=jax.ShapeDtypeStruct((B, D_H), jnp.float32),
    )(hf, sigc, keys_pad)
